# baseline (speedup 1.0000x reference)
import functools

import jax
import jax.numpy as jnp
import numpy as np
from jax import lax
from jax.experimental import pallas as pl

_B = 32
_PACK = 4
_L = _B * _PACK
_D = _B - 1
_LD = _D * _PACK
_LOWER = -3.0
_MIN_BIN = 0.01
_MIN_DERIV = 0.01
_RED_RANGE = 6.0 - _B * _MIN_BIN
_BLOCK = 512

_UPPER_CONST = float(
    np.float32(np.float32(_RED_RANGE) * np.float32(1.0)
               + np.float32(np.float32(_MIN_BIN) * np.float32(_B)))
    + np.float32(_LOWER))

_HIGH = lax.Precision.HIGHEST


def _seg_matrices():
    i = np.arange(_L)
    bd_lt = ((i[:, None] // _B == i[None, :] // _B)
             & (i[:, None] % _B <= i[None, :] % _B)).astype(np.float32)
    bd_ones = (i[:, None] // _B == i[None, :] // _B).astype(np.float32)
    e4 = (np.arange(_PACK)[:, None] == i[None, :] // _B).astype(np.float32)
    j = np.arange(_LD)
    m2 = (i[:, None] // _B == j[None, :] // _D).astype(np.float32)
    m31 = (j[:, None] // _D == i[None, :] // _B).astype(np.float32)
    c1 = (i[:, None] == _B * np.arange(_PACK)[None, :]).astype(np.float32)
    return (jnp.asarray(bd_lt), jnp.asarray(bd_ones), jnp.asarray(e4),
            jnp.asarray(m2), jnp.asarray(m31), jnp.asarray(c1))


def _mm(a, b, precision=_HIGH):
    return lax.dot_general(a, b, (((1,), (0,)), ((), ())),
                           preferred_element_type=jnp.float32,
                           precision=precision)


def _spline_body(x_ref, ubx_ref, uby_ref, ud_ref,
                 lt_ref, on_ref, e4_ref, m2_ref, m31_ref, c1_ref,
                 vals_ref, acc_ref):
    i = pl.program_id(0)
    x4 = x_ref[:, :]
    ubx = ubx_ref[:, :]
    uby = uby_ref[:, :]
    ud = ud_ref[:, :]
    bd_lt = lt_ref[:, :]
    bd_ones = on_ref[:, :]
    rows = ubx.shape[0]

    il = lax.broadcasted_iota(jnp.int32, (rows, _L), 1)
    kf = jnp.bitwise_and(il, _B - 1).astype(jnp.float32)

    xb = _mm(x4, e4_ref[:, :])

    ex = jnp.exp(ubx)
    ey = jnp.exp(uby)
    cs2 = _mm(jnp.concatenate([ex, ey], axis=0),
              jnp.concatenate([bd_lt, bd_ones], axis=1))
    kterm = _MIN_BIN * (kf + 1.0) + _LOWER

    def positions(cs, tot):
        return _RED_RANGE * (cs / tot) + kterm

    posx = positions(cs2[:rows, :_L], cs2[:rows, _L:])
    cqy = positions(cs2[rows:, :_L], cs2[rows:, _L:])

    in_range = jnp.logical_and(xb > _LOWER, xb < _UPPER_CONST)
    xs = jnp.clip(xb, _LOWER, _UPPER_CONST)

    c = (posx < xs).astype(jnp.float32)
    cn2 = _mm(c, jnp.concatenate([bd_ones, m2_ref[:, :]], axis=1),
              lax.Precision.DEFAULT)
    cnt = cn2[:, :_L]
    cnt124 = cn2[:, _L:]

    oh_lo = (kf == cnt - 1.0).astype(jnp.float32)
    oh_hi = (kf == cnt).astype(jnp.float32)
    sel = _mm(jnp.concatenate([posx * oh_lo, posx * oh_hi,
                               cqy * oh_lo, cqy * oh_hi], axis=0), bd_ones)
    lox = sel[:rows]
    upper_x = sel[rows:2 * rows]
    ylo = sel[2 * rows:3 * rows]
    yhi = sel[3 * rows:]
    first = cnt == 0.0
    last = cnt == jnp.float32(_B - 1)
    lower_x = jnp.where(first, jnp.float32(_LOWER), lox)
    lower_y = jnp.where(first, jnp.float32(_LOWER), ylo)
    upper_y = jnp.where(last, jnp.float32(_UPPER_CONST), yhi)

    il124 = lax.broadcasted_iota(jnp.int32, (rows, _LD), 1)
    j31 = (il124 - _D * (il124 // _D)).astype(jnp.float32)
    ohd_lo = (j31 == cnt124 - 1.0).astype(jnp.float32)
    ohd_hi = (j31 == cnt124).astype(jnp.float32)
    usel = _mm(jnp.concatenate([ud * ohd_lo, ud * ohd_hi], axis=0),
               m31_ref[:, :])
    ulo = usel[:rows]
    uhi = usel[rows:]

    def softplus(v):
        return jnp.maximum(v, 0.0) + jnp.log(1.0 + jnp.exp(-jnp.abs(v)))

    lower_d = jnp.where(first, 1.0, softplus(ulo) + _MIN_DERIV)
    upper_d = jnp.where(last, 1.0, softplus(uhi) + _MIN_DERIV)

    r_dx = 1.0 / (upper_x - lower_x)
    delta_y = upper_y - lower_y
    slope = delta_y * r_dx
    alpha = (xs - lower_x) * r_dx
    a2 = alpha * alpha
    beta = alpha * (1.0 - alpha)
    gamma = (1.0 - alpha) * (1.0 - alpha)
    eps = upper_d + lower_d - 2.0 * slope
    r_den = 1.0 / (slope + eps * beta)
    val_s = lower_y + delta_y * (slope * a2 + lower_d * beta) * r_den
    der_s = slope * slope * (upper_d * a2 + 2.0 * slope * beta
                             + lower_d * gamma) * (r_den * r_den)

    val = jnp.where(in_range, val_s, xb)
    der = jnp.where(in_range, der_s, 1.0)

    vals_ref[:, :] = _mm(val * (kf == 0.0).astype(jnp.float32), c1_ref[:, :])

    part = jnp.sum(jnp.log(jnp.abs(der)), axis=0, keepdims=True)
    part = jnp.sum(part, axis=1, keepdims=True) * (1.0 / _B)

    @pl.when(i == 0)
    def _init():
        acc_ref[:, :] = jnp.zeros((1, 1), jnp.float32)

    acc_ref[:, :] += part


@jax.jit
def kernel(x, unconst_bin_size_x, unconst_bin_size_y, unconst_derivs):
    n = x.shape[0]
    np4 = n // _PACK
    r = _BLOCK
    grid = np4 // r
    x4 = x.reshape(np4, _PACK)
    ubx = unconst_bin_size_x.reshape(np4, _L)
    uby = unconst_bin_size_y.reshape(np4, _L)
    ud = unconst_derivs.reshape(np4, _LD)
    mats = _seg_matrices()

    const_spec = [
        pl.BlockSpec(m.shape, lambda i: (0, 0)) for m in mats
    ]
    vals, acc = pl.pallas_call(
        _spline_body,
        grid=(grid,),
        in_specs=[
            pl.BlockSpec((r, _PACK), lambda i: (i, 0)),
            pl.BlockSpec((r, _L), lambda i: (i, 0)),
            pl.BlockSpec((r, _L), lambda i: (i, 0)),
            pl.BlockSpec((r, _LD), lambda i: (i, 0)),
        ] + const_spec,
        out_specs=[
            pl.BlockSpec((r, _PACK), lambda i: (i, 0)),
            pl.BlockSpec((1, 1), lambda i: (0, 0)),
        ],
        out_shape=[
            jax.ShapeDtypeStruct((np4, _PACK), jnp.float32),
            jax.ShapeDtypeStruct((1, 1), jnp.float32),
        ],
    )(x4, ubx, uby, ud, *mats)
    return vals.reshape(n), acc.reshape(())


# speedup vs baseline: 1.1203x; 1.1203x over previous
import functools

import jax
import jax.numpy as jnp
import numpy as np
from jax import lax
from jax.experimental import pallas as pl

_B = 32
_PACK = 4
_L = _B * _PACK
_D = _B - 1
_LD = _D * _PACK
_LOWER = -3.0
_MIN_BIN = 0.01
_MIN_DERIV = 0.01
_RED_RANGE = 6.0 - _B * _MIN_BIN
_BLOCK = 512

_UPPER_CONST = float(
    np.float32(np.float32(_RED_RANGE) * np.float32(1.0)
               + np.float32(np.float32(_MIN_BIN) * np.float32(_B)))
    + np.float32(_LOWER))

_HIGH = lax.Precision.HIGHEST


def _seg_matrices():
    i = np.arange(_L)
    bd_lt = ((i[:, None] // _B == i[None, :] // _B)
             & (i[:, None] % _B <= i[None, :] % _B)).astype(np.float32)
    bd_ones = (i[:, None] // _B == i[None, :] // _B).astype(np.float32)
    e4 = (np.arange(_PACK)[:, None] == i[None, :] // _B).astype(np.float32)
    j = np.arange(_LD)
    m2 = (i[:, None] // _B == j[None, :] // _D).astype(np.float32)
    m31 = (j[:, None] // _D == i[None, :] // _B).astype(np.float32)
    c1 = (i[:, None] == _B * np.arange(_PACK)[None, :]).astype(np.float32)
    return (jnp.asarray(bd_lt), jnp.asarray(bd_ones), jnp.asarray(e4),
            jnp.asarray(m2), jnp.asarray(m31), jnp.asarray(c1))


def _mm(a, b, precision=_HIGH):
    return lax.dot_general(a, b, (((1,), (0,)), ((), ())),
                           preferred_element_type=jnp.float32,
                           precision=precision)


def _spline_body(x_ref, ubx_ref, uby_ref, ud_ref,
                 lt_ref, on_ref, e4_ref, m2_ref, m31_ref, c1_ref,
                 vals_ref, acc_ref):
    i = pl.program_id(0)
    x4 = x_ref[:, :]
    ubx = ubx_ref[:, :]
    uby = uby_ref[:, :]
    ud = ud_ref[:, :]
    bd_lt = lt_ref[:, :]
    bd_ones = on_ref[:, :]
    rows = ubx.shape[0]

    il = lax.broadcasted_iota(jnp.int32, (rows, _L), 1)
    kf = jnp.bitwise_and(il, _B - 1).astype(jnp.float32)

    xb = _mm(x4, e4_ref[:, :])

    def positions(u):
        e = jnp.exp(u)
        cs = _mm(e, bd_lt)
        tot = _mm(e, bd_ones)
        return _RED_RANGE * (cs / tot) + _MIN_BIN * (kf + 1.0) + _LOWER

    posx = positions(ubx)
    cqy = positions(uby)

    in_range = jnp.logical_and(xb > _LOWER, xb < _UPPER_CONST)
    xs = jnp.clip(xb, _LOWER, _UPPER_CONST)

    c = (posx < xs).astype(jnp.float32)
    cnt = _mm(c, bd_ones, lax.Precision.DEFAULT)

    oh_lo = (kf == cnt - 1.0).astype(jnp.float32)
    oh_hi = (kf == cnt).astype(jnp.float32)
    lox = _mm(posx * oh_lo, bd_ones)
    upper_x = _mm(posx * oh_hi, bd_ones)
    ylo = _mm(cqy * oh_lo, bd_ones)
    yhi = _mm(cqy * oh_hi, bd_ones)
    first = cnt == 0.0
    last = cnt == jnp.float32(_B - 1)
    lower_x = jnp.where(first, jnp.float32(_LOWER), lox)
    lower_y = jnp.where(first, jnp.float32(_LOWER), ylo)
    upper_y = jnp.where(last, jnp.float32(_UPPER_CONST), yhi)

    il124 = lax.broadcasted_iota(jnp.int32, (rows, _LD), 1)
    j31 = (il124 - _D * (il124 // _D)).astype(jnp.float32)
    cnt124 = _mm(c, m2_ref[:, :], lax.Precision.DEFAULT)
    ohd_lo = (j31 == cnt124 - 1.0).astype(jnp.float32)
    ohd_hi = (j31 == cnt124).astype(jnp.float32)
    ulo = _mm(ud * ohd_lo, m31_ref[:, :])
    uhi = _mm(ud * ohd_hi, m31_ref[:, :])

    def softplus(v):
        return jnp.maximum(v, 0.0) + jnp.log(1.0 + jnp.exp(-jnp.abs(v)))

    lower_d = jnp.where(first, 1.0, softplus(ulo) + _MIN_DERIV)
    upper_d = jnp.where(last, 1.0, softplus(uhi) + _MIN_DERIV)

    r_dx = 1.0 / (upper_x - lower_x)
    delta_y = upper_y - lower_y
    slope = delta_y * r_dx
    alpha = (xs - lower_x) * r_dx
    a2 = alpha * alpha
    beta = alpha * (1.0 - alpha)
    gamma = (1.0 - alpha) * (1.0 - alpha)
    eps = upper_d + lower_d - 2.0 * slope
    r_den = 1.0 / (slope + eps * beta)
    val_s = lower_y + delta_y * (slope * a2 + lower_d * beta) * r_den
    der_s = slope * slope * (upper_d * a2 + 2.0 * slope * beta
                             + lower_d * gamma) * (r_den * r_den)

    val = jnp.where(in_range, val_s, xb)
    der = jnp.where(in_range, der_s, 1.0)

    vals_ref[:, :] = _mm(val * (kf == 0.0).astype(jnp.float32), c1_ref[:, :])

    part = jnp.sum(jnp.log(jnp.abs(der)), axis=0, keepdims=True)
    part = jnp.sum(part, axis=1, keepdims=True) * (1.0 / _B)

    @pl.when(i == 0)
    def _init():
        acc_ref[:, :] = jnp.zeros((1, 1), jnp.float32)

    acc_ref[:, :] += part


@jax.jit
def kernel(x, unconst_bin_size_x, unconst_bin_size_y, unconst_derivs):
    n = x.shape[0]
    np4 = n // _PACK
    r = _BLOCK
    grid = np4 // r
    x4 = x.reshape(np4, _PACK)
    ubx = unconst_bin_size_x.reshape(np4, _L)
    uby = unconst_bin_size_y.reshape(np4, _L)
    ud = unconst_derivs.reshape(np4, _LD)
    mats = _seg_matrices()

    const_spec = [
        pl.BlockSpec(m.shape, lambda i: (0, 0)) for m in mats
    ]
    vals, acc = pl.pallas_call(
        _spline_body,
        grid=(grid,),
        in_specs=[
            pl.BlockSpec((r, _PACK), lambda i: (i, 0)),
            pl.BlockSpec((r, _L), lambda i: (i, 0)),
            pl.BlockSpec((r, _L), lambda i: (i, 0)),
            pl.BlockSpec((r, _LD), lambda i: (i, 0)),
        ] + const_spec,
        out_specs=[
            pl.BlockSpec((r, _PACK), lambda i: (i, 0)),
            pl.BlockSpec((1, 1), lambda i: (0, 0)),
        ],
        out_shape=[
            jax.ShapeDtypeStruct((np4, _PACK), jnp.float32),
            jax.ShapeDtypeStruct((1, 1), jnp.float32),
        ],
    )(x4, ubx, uby, ud, *mats)
    return vals.reshape(n), acc.reshape(())


# speedup vs baseline: 1.2381x; 1.1051x over previous
import functools

import jax
import jax.numpy as jnp
import numpy as np
from jax import lax
from jax.experimental import pallas as pl

_B = 32
_PACK = 4
_L = _B * _PACK
_D = _B - 1
_LD = _D * _PACK
_LOWER = -3.0
_MIN_BIN = 0.01
_MIN_DERIV = 0.01
_RED_RANGE = 6.0 - _B * _MIN_BIN
_BLOCK = 512

_UPPER_CONST = float(
    np.float32(np.float32(_RED_RANGE) * np.float32(1.0)
               + np.float32(np.float32(_MIN_BIN) * np.float32(_B)))
    + np.float32(_LOWER))

_HI = lax.Precision.HIGHEST
_LO = lax.Precision.DEFAULT


def _seg_matrices():
    i = np.arange(_L)
    seg = i[:, None] // _B == i[None, :] // _B
    bd_lt = (seg & (i[:, None] % _B <= i[None, :] % _B)).astype(np.float32)
    bd_ones = seg.astype(np.float32)
    lts = np.concatenate([bd_lt, bd_ones], axis=1)
    e4 = (np.arange(_PACK)[:, None] == i[None, :] // _B).astype(np.float32)
    j = np.arange(_LD)
    m2 = (i[:, None] // _B == j[None, :] // _D).astype(np.float32)
    cn2 = np.concatenate([bd_ones, m2], axis=1)
    m31 = (j[:, None] // _D == i[None, :] // _B).astype(np.float32)
    c1 = (i[:, None] == _B * np.arange(_PACK)[None, :]).astype(np.float32)
    return (jnp.asarray(lts), jnp.asarray(bd_ones), jnp.asarray(e4),
            jnp.asarray(cn2), jnp.asarray(m31), jnp.asarray(c1))


def _mm(a, b, precision=_HI):
    return lax.dot_general(a, b, (((1,), (0,)), ((), ())),
                           preferred_element_type=jnp.float32,
                           precision=precision)


def _spline_body(x_ref, ubx_ref, uby_ref, ud_ref,
                 lts_ref, on_ref, e4_ref, cn2_ref, m31_ref, c1_ref,
                 vals_ref, acc_ref):
    i = pl.program_id(0)
    x4 = x_ref[:, :]
    ubx = ubx_ref[:, :]
    uby = uby_ref[:, :]
    ud = ud_ref[:, :]
    bd_ones = on_ref[:, :]
    rows = ubx.shape[0]

    il = lax.broadcasted_iota(jnp.int32, (rows, _L), 1)
    kf = jnp.bitwise_and(il, _B - 1).astype(jnp.float32)
    kterm = _MIN_BIN * (kf + 1.0) + _LOWER

    xb = _mm(x4, e4_ref[:, :])

    ex = jnp.exp(ubx)
    ey = jnp.exp(uby)
    cs2x = _mm(ex, lts_ref[:, :])
    cs2y = _mm(ey, lts_ref[:, :])
    totx = cs2x[:, _L:]
    toty = cs2y[:, _L:]
    posx = _RED_RANGE * (cs2x[:, :_L] / totx) + kterm
    cqy = _RED_RANGE * (cs2y[:, :_L] / toty) + kterm

    in_range = jnp.logical_and(xb > _LOWER, xb < _UPPER_CONST)
    xs = jnp.clip(xb, _LOWER, _UPPER_CONST)

    c = (posx < xs).astype(jnp.float32)
    cn2 = _mm(c, cn2_ref[:, :], _LO)
    cnt = cn2[:, :_L]
    cnt124 = cn2[:, _L:]

    oh_lo = (kf == cnt - 1.0).astype(jnp.float32)
    oh_hi = (kf == cnt).astype(jnp.float32)
    first = cnt == 0.0
    last = cnt == jnp.float32(_B - 1)

    lox = _mm(posx * oh_lo, bd_ones)
    ylo = _mm(cqy * oh_lo, bd_ones)
    lower_x = jnp.where(first, jnp.float32(_LOWER), lox)
    lower_y = jnp.where(first, jnp.float32(_LOWER), ylo)

    dex = _mm(ex * oh_hi, bd_ones, _LO)
    dey = _mm(ey * oh_hi, bd_ones, _LO)
    delta_x = _RED_RANGE * (dex / totx) + _MIN_BIN
    delta_y = jnp.where(last, jnp.float32(_UPPER_CONST) - lower_y,
                        _RED_RANGE * (dey / toty) + _MIN_BIN)

    il124 = lax.broadcasted_iota(jnp.int32, (rows, _LD), 1)
    j31 = (il124 - _D * (il124 // _D)).astype(jnp.float32)
    ohd_lo = (j31 == cnt124 - 1.0).astype(jnp.float32)
    ohd_hi = (j31 == cnt124).astype(jnp.float32)
    ulo = _mm(ud * ohd_lo, m31_ref[:, :], _LO)
    uhi = _mm(ud * ohd_hi, m31_ref[:, :], _LO)

    def softplus(v):
        return jnp.maximum(v, 0.0) + jnp.log(1.0 + jnp.exp(-jnp.abs(v)))

    lower_d = jnp.where(first, 1.0, softplus(ulo) + _MIN_DERIV)
    upper_d = jnp.where(last, 1.0, softplus(uhi) + _MIN_DERIV)

    r_dx = 1.0 / delta_x
    slope = delta_y * r_dx
    alpha = jnp.clip((xs - lower_x) * r_dx, 0.0, 1.0)
    a2 = alpha * alpha
    om = 1.0 - alpha
    beta = alpha * om
    gamma = om * om
    eps = upper_d + lower_d - 2.0 * slope
    r_den = 1.0 / (slope + eps * beta)
    val_s = lower_y + delta_y * (slope * a2 + lower_d * beta) * r_den
    der_s = slope * slope * (upper_d * a2 + 2.0 * slope * beta
                             + lower_d * gamma) * (r_den * r_den)

    val = jnp.where(in_range, val_s, xb)
    der = jnp.where(in_range, der_s, 1.0)

    vals_ref[:, :] = _mm(val * (kf == 0.0).astype(jnp.float32), c1_ref[:, :])

    part = jnp.sum(jnp.log(jnp.abs(der)), axis=0, keepdims=True)
    part = jnp.sum(part, axis=1, keepdims=True) * (1.0 / _B)

    @pl.when(i == 0)
    def _init():
        acc_ref[:, :] = jnp.zeros((1, 1), jnp.float32)

    acc_ref[:, :] += part


@jax.jit
def kernel(x, unconst_bin_size_x, unconst_bin_size_y, unconst_derivs):
    n = x.shape[0]
    np4 = n // _PACK
    r = _BLOCK
    grid = np4 // r
    x4 = x.reshape(np4, _PACK)
    ubx = unconst_bin_size_x.reshape(np4, _L)
    uby = unconst_bin_size_y.reshape(np4, _L)
    ud = unconst_derivs.reshape(np4, _LD)
    mats = _seg_matrices()

    const_spec = [
        pl.BlockSpec(m.shape, lambda i: (0, 0)) for m in mats
    ]
    vals, acc = pl.pallas_call(
        _spline_body,
        grid=(grid,),
        in_specs=[
            pl.BlockSpec((r, _PACK), lambda i: (i, 0)),
            pl.BlockSpec((r, _L), lambda i: (i, 0)),
            pl.BlockSpec((r, _L), lambda i: (i, 0)),
            pl.BlockSpec((r, _LD), lambda i: (i, 0)),
        ] + const_spec,
        out_specs=[
            pl.BlockSpec((r, _PACK), lambda i: (i, 0)),
            pl.BlockSpec((1, 1), lambda i: (0, 0)),
        ],
        out_shape=[
            jax.ShapeDtypeStruct((np4, _PACK), jnp.float32),
            jax.ShapeDtypeStruct((1, 1), jnp.float32),
        ],
    )(x4, ubx, uby, ud, *mats)
    return vals.reshape(n), acc.reshape(())


# speedup vs baseline: 1.2740x; 1.0290x over previous
import functools

import jax
import jax.numpy as jnp
import numpy as np
from jax import lax
from jax.experimental import pallas as pl

_B = 32
_PACK = 4
_L = _B * _PACK
_D = _B - 1
_LD = _D * _PACK
_LOWER = -3.0
_MIN_BIN = 0.01
_MIN_DERIV = 0.01
_RED_RANGE = 6.0 - _B * _MIN_BIN
_BLOCK = 1024

_UPPER_CONST = float(
    np.float32(np.float32(_RED_RANGE) * np.float32(1.0)
               + np.float32(np.float32(_MIN_BIN) * np.float32(_B)))
    + np.float32(_LOWER))

_HI = lax.Precision.HIGHEST
_LO = lax.Precision.DEFAULT


def _seg_matrices():
    i = np.arange(_L)
    seg = i[:, None] // _B == i[None, :] // _B
    bd_lt = (seg & (i[:, None] % _B <= i[None, :] % _B)).astype(np.float32)
    bd_ones = seg.astype(np.float32)
    lts = np.concatenate([bd_lt, bd_ones], axis=1)
    e4 = (np.arange(_PACK)[:, None] == i[None, :] // _B).astype(np.float32)
    j = np.arange(_LD)
    m2 = (i[:, None] // _B == j[None, :] // _D).astype(np.float32)
    cn2 = np.concatenate([bd_ones, m2], axis=1)
    m31 = (j[:, None] // _D == i[None, :] // _B).astype(np.float32)
    c1 = (i[:, None] == _B * np.arange(_PACK)[None, :]).astype(np.float32)
    return (jnp.asarray(lts), jnp.asarray(bd_ones), jnp.asarray(e4),
            jnp.asarray(cn2), jnp.asarray(m31), jnp.asarray(c1))


def _mm(a, b, precision=_HI):
    return lax.dot_general(a, b, (((1,), (0,)), ((), ())),
                           preferred_element_type=jnp.float32,
                           precision=precision)


def _spline_body(x_ref, ubx_ref, uby_ref, ud_ref,
                 lts_ref, on_ref, e4_ref, cn2_ref, m31_ref, c1_ref,
                 vals_ref, acc_ref):
    i = pl.program_id(0)
    x4 = x_ref[:, :]
    ubx = ubx_ref[:, :]
    uby = uby_ref[:, :]
    ud = ud_ref[:, :]
    bd_ones = on_ref[:, :]
    rows = ubx.shape[0]

    il = lax.broadcasted_iota(jnp.int32, (rows, _L), 1)
    kf = jnp.bitwise_and(il, _B - 1).astype(jnp.float32)
    kterm = _MIN_BIN * (kf + 1.0) + _LOWER

    xb = _mm(x4, e4_ref[:, :])

    ex = jnp.exp(ubx)
    ey = jnp.exp(uby)
    cs2x = _mm(ex, lts_ref[:, :])
    cs2y = _mm(ey, lts_ref[:, :])
    totx = cs2x[:, _L:]
    toty = cs2y[:, _L:]
    posx = _RED_RANGE * (cs2x[:, :_L] / totx) + kterm
    cqy = _RED_RANGE * (cs2y[:, :_L] / toty) + kterm

    in_range = jnp.logical_and(xb > _LOWER, xb < _UPPER_CONST)
    xs = jnp.clip(xb, _LOWER, _UPPER_CONST)

    c = (posx < xs).astype(jnp.float32)
    cn2 = _mm(c, cn2_ref[:, :], _LO)
    cnt = cn2[:, :_L]
    cnt124 = cn2[:, _L:]

    oh_lo = (kf == cnt - 1.0).astype(jnp.float32)
    oh_hi = (kf == cnt).astype(jnp.float32)
    first = cnt == 0.0
    last = cnt == jnp.float32(_B - 1)

    lox = _mm(posx * oh_lo, bd_ones)
    ylo = _mm(cqy * oh_lo, bd_ones)
    lower_x = jnp.where(first, jnp.float32(_LOWER), lox)
    lower_y = jnp.where(first, jnp.float32(_LOWER), ylo)

    dex = _mm(ex * oh_hi, bd_ones, _LO)
    dey = _mm(ey * oh_hi, bd_ones, _LO)
    delta_x = _RED_RANGE * (dex / totx) + _MIN_BIN
    delta_y = jnp.where(last, jnp.float32(_UPPER_CONST) - lower_y,
                        _RED_RANGE * (dey / toty) + _MIN_BIN)

    il124 = lax.broadcasted_iota(jnp.int32, (rows, _LD), 1)
    j31 = (il124 - _D * (il124 // _D)).astype(jnp.float32)
    ohd_lo = (j31 == cnt124 - 1.0).astype(jnp.float32)
    ohd_hi = (j31 == cnt124).astype(jnp.float32)
    ulo = _mm(ud * ohd_lo, m31_ref[:, :], _LO)
    uhi = _mm(ud * ohd_hi, m31_ref[:, :], _LO)

    def softplus(v):
        return jnp.maximum(v, 0.0) + jnp.log(1.0 + jnp.exp(-jnp.abs(v)))

    lower_d = jnp.where(first, 1.0, softplus(ulo) + _MIN_DERIV)
    upper_d = jnp.where(last, 1.0, softplus(uhi) + _MIN_DERIV)

    r_dx = 1.0 / delta_x
    slope = delta_y * r_dx
    alpha = jnp.clip((xs - lower_x) * r_dx, 0.0, 1.0)
    a2 = alpha * alpha
    om = 1.0 - alpha
    beta = alpha * om
    gamma = om * om
    eps = upper_d + lower_d - 2.0 * slope
    r_den = 1.0 / (slope + eps * beta)
    val_s = lower_y + delta_y * (slope * a2 + lower_d * beta) * r_den
    der_s = slope * slope * (upper_d * a2 + 2.0 * slope * beta
                             + lower_d * gamma) * (r_den * r_den)

    val = jnp.where(in_range, val_s, xb)
    der = jnp.where(in_range, der_s, 1.0)

    vals_ref[:, :] = _mm(val * (kf == 0.0).astype(jnp.float32), c1_ref[:, :])

    part = jnp.sum(jnp.log(jnp.abs(der)), axis=0, keepdims=True)
    part = jnp.sum(part, axis=1, keepdims=True) * (1.0 / _B)

    @pl.when(i == 0)
    def _init():
        acc_ref[:, :] = jnp.zeros((1, 1), jnp.float32)

    acc_ref[:, :] += part


@jax.jit
def kernel(x, unconst_bin_size_x, unconst_bin_size_y, unconst_derivs):
    n = x.shape[0]
    np4 = n // _PACK
    r = _BLOCK
    grid = np4 // r
    x4 = x.reshape(np4, _PACK)
    ubx = unconst_bin_size_x.reshape(np4, _L)
    uby = unconst_bin_size_y.reshape(np4, _L)
    ud = unconst_derivs.reshape(np4, _LD)
    mats = _seg_matrices()

    const_spec = [
        pl.BlockSpec(m.shape, lambda i: (0, 0)) for m in mats
    ]
    vals, acc = pl.pallas_call(
        _spline_body,
        grid=(grid,),
        in_specs=[
            pl.BlockSpec((r, _PACK), lambda i: (i, 0)),
            pl.BlockSpec((r, _L), lambda i: (i, 0)),
            pl.BlockSpec((r, _L), lambda i: (i, 0)),
            pl.BlockSpec((r, _LD), lambda i: (i, 0)),
        ] + const_spec,
        out_specs=[
            pl.BlockSpec((r, _PACK), lambda i: (i, 0)),
            pl.BlockSpec((1, 1), lambda i: (0, 0)),
        ],
        out_shape=[
            jax.ShapeDtypeStruct((np4, _PACK), jnp.float32),
            jax.ShapeDtypeStruct((1, 1), jnp.float32),
        ],
    )(x4, ubx, uby, ud, *mats)
    return vals.reshape(n), acc.reshape(())


# speedup vs baseline: 1.2826x; 1.0067x over previous
import functools

import jax
import jax.numpy as jnp
import numpy as np
from jax import lax
from jax.experimental import pallas as pl

_B = 32
_PACK = 4
_L = _B * _PACK
_D = _B - 1
_LD = _D * _PACK
_LOWER = -3.0
_MIN_BIN = 0.01
_MIN_DERIV = 0.01
_RED_RANGE = 6.0 - _B * _MIN_BIN
_BLOCK = 2048

_UPPER_CONST = float(
    np.float32(np.float32(_RED_RANGE) * np.float32(1.0)
               + np.float32(np.float32(_MIN_BIN) * np.float32(_B)))
    + np.float32(_LOWER))

_HI = lax.Precision.HIGHEST
_LO = lax.Precision.DEFAULT


def _seg_matrices():
    i = np.arange(_L)
    seg = i[:, None] // _B == i[None, :] // _B
    bd_lt = (seg & (i[:, None] % _B <= i[None, :] % _B)).astype(np.float32)
    bd_ones = seg.astype(np.float32)
    lts = np.concatenate([bd_lt, bd_ones], axis=1)
    e4 = (np.arange(_PACK)[:, None] == i[None, :] // _B).astype(np.float32)
    j = np.arange(_LD)
    m2 = (i[:, None] // _B == j[None, :] // _D).astype(np.float32)
    cn2 = np.concatenate([bd_ones, m2], axis=1)
    m31 = (j[:, None] // _D == i[None, :] // _B).astype(np.float32)
    c1 = (i[:, None] == _B * np.arange(_PACK)[None, :]).astype(np.float32)
    return (jnp.asarray(lts), jnp.asarray(bd_ones), jnp.asarray(e4),
            jnp.asarray(cn2), jnp.asarray(m31), jnp.asarray(c1))


def _mm(a, b, precision=_HI):
    return lax.dot_general(a, b, (((1,), (0,)), ((), ())),
                           preferred_element_type=jnp.float32,
                           precision=precision)


def _spline_body(x_ref, ubx_ref, uby_ref, ud_ref,
                 lts_ref, on_ref, e4_ref, cn2_ref, m31_ref, c1_ref,
                 vals_ref, acc_ref):
    i = pl.program_id(0)
    x4 = x_ref[:, :]
    ubx = ubx_ref[:, :]
    uby = uby_ref[:, :]
    ud = ud_ref[:, :]
    bd_ones = on_ref[:, :]
    rows = ubx.shape[0]

    il = lax.broadcasted_iota(jnp.int32, (rows, _L), 1)
    kf = jnp.bitwise_and(il, _B - 1).astype(jnp.float32)
    kterm = _MIN_BIN * (kf + 1.0) + _LOWER

    xb = _mm(x4, e4_ref[:, :])

    ex = jnp.exp(ubx)
    ey = jnp.exp(uby)
    cs2x = _mm(ex, lts_ref[:, :])
    cs2y = _mm(ey, lts_ref[:, :])
    totx = cs2x[:, _L:]
    toty = cs2y[:, _L:]
    posx = _RED_RANGE * (cs2x[:, :_L] / totx) + kterm
    cqy = _RED_RANGE * (cs2y[:, :_L] / toty) + kterm

    in_range = jnp.logical_and(xb > _LOWER, xb < _UPPER_CONST)
    xs = jnp.clip(xb, _LOWER, _UPPER_CONST)

    c = (posx < xs).astype(jnp.float32)
    cn2 = _mm(c, cn2_ref[:, :], _LO)
    cnt = cn2[:, :_L]
    cnt124 = cn2[:, _L:]

    oh_lo = (kf == cnt - 1.0).astype(jnp.float32)
    oh_hi = (kf == cnt).astype(jnp.float32)
    first = cnt == 0.0
    last = cnt == jnp.float32(_B - 1)

    lox = _mm(posx * oh_lo, bd_ones)
    ylo = _mm(cqy * oh_lo, bd_ones)
    lower_x = jnp.where(first, jnp.float32(_LOWER), lox)
    lower_y = jnp.where(first, jnp.float32(_LOWER), ylo)

    dex = _mm(ex * oh_hi, bd_ones, _LO)
    dey = _mm(ey * oh_hi, bd_ones, _LO)
    delta_x = _RED_RANGE * (dex / totx) + _MIN_BIN
    delta_y = jnp.where(last, jnp.float32(_UPPER_CONST) - lower_y,
                        _RED_RANGE * (dey / toty) + _MIN_BIN)

    il124 = lax.broadcasted_iota(jnp.int32, (rows, _LD), 1)
    j31 = (il124 - _D * (il124 // _D)).astype(jnp.float32)
    ohd_lo = (j31 == cnt124 - 1.0).astype(jnp.float32)
    ohd_hi = (j31 == cnt124).astype(jnp.float32)
    ulo = _mm(ud * ohd_lo, m31_ref[:, :], _LO)
    uhi = _mm(ud * ohd_hi, m31_ref[:, :], _LO)

    def softplus(v):
        return jnp.maximum(v, 0.0) + jnp.log(1.0 + jnp.exp(-jnp.abs(v)))

    lower_d = jnp.where(first, 1.0, softplus(ulo) + _MIN_DERIV)
    upper_d = jnp.where(last, 1.0, softplus(uhi) + _MIN_DERIV)

    r_dx = 1.0 / delta_x
    slope = delta_y * r_dx
    alpha = jnp.clip((xs - lower_x) * r_dx, 0.0, 1.0)
    a2 = alpha * alpha
    om = 1.0 - alpha
    beta = alpha * om
    gamma = om * om
    eps = upper_d + lower_d - 2.0 * slope
    r_den = 1.0 / (slope + eps * beta)
    val_s = lower_y + delta_y * (slope * a2 + lower_d * beta) * r_den
    der_s = slope * slope * (upper_d * a2 + 2.0 * slope * beta
                             + lower_d * gamma) * (r_den * r_den)

    val = jnp.where(in_range, val_s, xb)
    der = jnp.where(in_range, der_s, 1.0)

    vals_ref[:, :] = _mm(val * (kf == 0.0).astype(jnp.float32), c1_ref[:, :])

    part = jnp.sum(jnp.log(jnp.abs(der)), axis=0, keepdims=True)
    part = jnp.sum(part, axis=1, keepdims=True) * (1.0 / _B)

    @pl.when(i == 0)
    def _init():
        acc_ref[:, :] = jnp.zeros((1, 1), jnp.float32)

    acc_ref[:, :] += part


@jax.jit
def kernel(x, unconst_bin_size_x, unconst_bin_size_y, unconst_derivs):
    n = x.shape[0]
    np4 = n // _PACK
    r = _BLOCK
    grid = np4 // r
    x4 = x.reshape(np4, _PACK)
    ubx = unconst_bin_size_x.reshape(np4, _L)
    uby = unconst_bin_size_y.reshape(np4, _L)
    ud = unconst_derivs.reshape(np4, _LD)
    mats = _seg_matrices()

    const_spec = [
        pl.BlockSpec(m.shape, lambda i: (0, 0)) for m in mats
    ]
    vals, acc = pl.pallas_call(
        _spline_body,
        grid=(grid,),
        in_specs=[
            pl.BlockSpec((r, _PACK), lambda i: (i, 0)),
            pl.BlockSpec((r, _L), lambda i: (i, 0)),
            pl.BlockSpec((r, _L), lambda i: (i, 0)),
            pl.BlockSpec((r, _LD), lambda i: (i, 0)),
        ] + const_spec,
        out_specs=[
            pl.BlockSpec((r, _PACK), lambda i: (i, 0)),
            pl.BlockSpec((1, 1), lambda i: (0, 0)),
        ],
        out_shape=[
            jax.ShapeDtypeStruct((np4, _PACK), jnp.float32),
            jax.ShapeDtypeStruct((1, 1), jnp.float32),
        ],
    )(x4, ubx, uby, ud, *mats)
    return vals.reshape(n), acc.reshape(())
